# Initial kernel scaffold; baseline (speedup 1.0000x reference)
import jax
import jax.numpy as jnp
from jax.experimental import pallas as pl


def kernel(xyz, params):
    raise NotImplementedError("write your pallas kernel here")


# speedup vs baseline: 12.4522x; 12.4522x over previous
import numpy as np
import jax
import jax.numpy as jnp
from jax import lax
from jax.experimental import pallas as pl
from jax.experimental.pallas import tpu as pltpu

_BN = float(1.0 / np.sqrt(1.0 + 1e-5))
_BIGF = 1e30
_IBIG = 1e9
_NS = 32


def _prep(p):
    return (p['W'].T, p['b'][None, :], p['gamma'][None, :],
            p['beta'][None, :])


def _fps_call(cx, cy, cz, npoint):
    B, N = cx.shape

    def body(x_ref, y_ref, z_ref, ox_ref, oy_ref, oz_ref):
        x = x_ref[...]
        y = y_ref[...]
        z = z_ref[...]
        lane = lax.broadcasted_iota(jnp.int32, (B, N), 1)
        lane_o = lax.broadcasted_iota(jnp.int32, (B, npoint), 1)

        def step(i, carry):
            dist, far, ax, ay, az = carry
            sel = lane == far
            cxv = jnp.sum(jnp.where(sel, x, 0.0), axis=1, keepdims=True)
            cyv = jnp.sum(jnp.where(sel, y, 0.0), axis=1, keepdims=True)
            czv = jnp.sum(jnp.where(sel, z, 0.0), axis=1, keepdims=True)
            hit = lane_o == i
            ax = jnp.where(hit, cxv, ax)
            ay = jnp.where(hit, cyv, ay)
            az = jnp.where(hit, czv, az)
            dx = x - cxv
            dy = y - cyv
            dz = z - czv
            d = dx * dx + dy * dy + dz * dz
            dist = jnp.minimum(dist, d)
            m = jnp.max(dist, axis=1, keepdims=True)
            far = jnp.min(jnp.where(dist == m, lane, jnp.int32(2**30)),
                          axis=1, keepdims=True)
            return dist, far, ax, ay, az

        z0 = jnp.zeros((B, npoint), jnp.float32)
        carry = (jnp.full((B, N), 1e10, jnp.float32),
                 jnp.zeros((B, 1), jnp.int32), z0, z0, z0)
        _, _, ax, ay, az = lax.fori_loop(0, npoint, step, carry)
        ox_ref[...] = ax
        oy_ref[...] = ay
        oz_ref[...] = az

    outs = [jax.ShapeDtypeStruct((B, npoint), jnp.float32)] * 3
    return pl.pallas_call(body, out_shape=outs)(cx, cy, cz)


def _ball_call(px, py, pz, c, r2, s_tile):
    B, _, N = px.shape
    S = c.shape[1]
    grid = (B, S // s_tile)

    def body(x_ref, y_ref, z_ref, c_ref, o_ref, key_ref):
        x = x_ref[...].reshape(1, N)
        y = y_ref[...].reshape(1, N)
        z = z_ref[...].reshape(1, N)
        cc = c_ref[...].reshape(s_tile, 3)
        cx = cc[:, 0:1]
        cy = cc[:, 1:2]
        cz = cc[:, 2:3]
        sn = x * x + y * y + z * z
        cn = cx * cx + cy * cy + cz * cz
        xb = x.astype(jnp.bfloat16).astype(jnp.float32)
        yb = y.astype(jnp.bfloat16).astype(jnp.float32)
        zb = z.astype(jnp.bfloat16).astype(jnp.float32)
        cxb = cx.astype(jnp.bfloat16).astype(jnp.float32)
        cyb = cy.astype(jnp.bfloat16).astype(jnp.float32)
        czb = cz.astype(jnp.bfloat16).astype(jnp.float32)
        dot = cxb * xb + cyb * yb + czb * zb
        d2 = (-2.0 * dot + cn) + sn
        lane = lax.broadcasted_iota(jnp.int32, (s_tile, N), 1
                                    ).astype(jnp.float32)
        key_ref[...] = jnp.where(d2 <= r2, lane, _BIGF)
        lane_k = lax.broadcasted_iota(jnp.int32, (s_tile, _NS), 1)

        def step(k, carry):
            acc, first = carry
            key = key_ref[...]
            m = jnp.min(key, axis=1, keepdims=True)
            first = jnp.where(k == 0, m, first)
            val = jnp.where(m >= _BIGF, first, m)
            acc = jnp.where(lane_k == k, val.astype(jnp.int32), acc)
            key_ref[...] = jnp.where(key == m, _BIGF, key)
            return acc, first

        acc, _ = lax.fori_loop(
            0, _NS, step,
            (jnp.zeros((s_tile, _NS), jnp.int32),
             jnp.zeros((s_tile, 1), jnp.float32)))
        o_ref[...] = acc.reshape(1, s_tile, _NS)

    return pl.pallas_call(
        body, grid=grid,
        in_specs=[pl.BlockSpec((1, 1, N), lambda b, s: (b, 0, 0)),
                  pl.BlockSpec((1, 1, N), lambda b, s: (b, 0, 0)),
                  pl.BlockSpec((1, 1, N), lambda b, s: (b, 0, 0)),
                  pl.BlockSpec((1, s_tile, 3), lambda b, s: (b, s, 0))],
        out_specs=pl.BlockSpec((1, s_tile, _NS), lambda b, s: (b, s, 0)),
        out_shape=jax.ShapeDtypeStruct((B, S, _NS), jnp.int32),
        scratch_shapes=[pltpu.VMEM((s_tile, N), jnp.float32)],
    )(px, py, pz, c)


def _sa_mlp_call(g, c, wbs, cin, s_tile):
    B, S, ns, Dp = g.shape
    cout = wbs[-1][0].shape[1]
    grid = (B, S // s_tile)
    nw = len(wbs)

    def body(g_ref, c_ref, *refs):
        w_refs = refs[:4 * nw]
        o_ref = refs[4 * nw]
        gv = g_ref[...].reshape(s_tile, ns, Dp)
        cc = c_ref[...].reshape(s_tile, 3)
        cpad = jnp.concatenate(
            [cc, jnp.zeros((s_tile, cin - 3), jnp.float32)], axis=1)
        h = (gv[:, :, 0:cin] - cpad[:, None, :]).reshape(s_tile * ns, cin)
        for j in range(nw):
            w, b, gm, bt = (w_refs[4 * j][...], w_refs[4 * j + 1][...],
                            w_refs[4 * j + 2][...], w_refs[4 * j + 3][...])
            t = jnp.dot(h, w, preferred_element_type=jnp.float32) + b
            h = jnp.maximum((t * _BN) * gm + bt, 0.0)
        h = jnp.max(h.reshape(s_tile, ns, cout), axis=1)
        o_ref[...] = h.reshape(1, s_tile, cout)

    in_specs = [pl.BlockSpec((1, s_tile, ns, Dp), lambda b, s: (b, s, 0, 0)),
                pl.BlockSpec((1, s_tile, 3), lambda b, s: (b, s, 0))]
    args = [g, c]
    for wb in wbs:
        for t_ in wb:
            in_specs.append(
                pl.BlockSpec(t_.shape, lambda *_, n=t_.ndim: (0,) * n))
            args.append(t_)
    return pl.pallas_call(
        body, grid=grid, in_specs=in_specs,
        out_specs=pl.BlockSpec((1, s_tile, cout), lambda b, s: (b, s, 0)),
        out_shape=jax.ShapeDtypeStruct((B, S, cout), jnp.float32))(*args)


def _fp_call(x1, x2c, p2, pts1, wbs, n_tile):
    B, N, _ = x1.shape
    S, C2 = p2.shape[1], p2.shape[2]
    C1 = 0 if pts1 is None else pts1.shape[2]
    cout = wbs[-1][0].shape[1]
    nw = len(wbs)
    grid = (B, N // n_tile)

    def body(*refs):
        x1_ref, xx_ref, xy_ref, xz_ref, p2_ref = refs[:5]
        k = 5
        p1_ref = None
        if C1:
            p1_ref = refs[5]
            k = 6
        w_refs = refs[k:k + 4 * nw]
        o_ref = refs[k + 4 * nw]
        a = x1_ref[...].reshape(n_tile, 3)
        a0 = a[:, 0:1]
        a1 = a[:, 1:2]
        a2 = a[:, 2:3]
        bx = xx_ref[...].reshape(1, S)
        by = xy_ref[...].reshape(1, S)
        bz = xz_ref[...].reshape(1, S)
        n1 = a0 * a0 + a1 * a1 + a2 * a2
        n2 = bx * bx + by * by + bz * bz
        bxb = bx.astype(jnp.bfloat16).astype(jnp.float32)
        byb = by.astype(jnp.bfloat16).astype(jnp.float32)
        bzb = bz.astype(jnp.bfloat16).astype(jnp.float32)
        a0b = a0.astype(jnp.bfloat16).astype(jnp.float32)
        a1b = a1.astype(jnp.bfloat16).astype(jnp.float32)
        a2b = a2.astype(jnp.bfloat16).astype(jnp.float32)
        dot = a0b * bxb + a1b * byb + a2b * bzb
        d2 = (-2.0 * dot + n1) + n2
        lane = lax.broadcasted_iota(jnp.int32, (n_tile, S), 1
                                    ).astype(jnp.float32)
        key = d2
        recips = []
        sels = []
        for _ in range(3):
            m = jnp.min(key, axis=1, keepdims=True)
            col = jnp.min(jnp.where(key == m, lane, _BIGF), axis=1,
                          keepdims=True)
            sel = lane == col
            recips.append(1.0 / (m + 1e-8))
            sels.append(sel)
            key = jnp.where(sel, _BIGF, key)
        norm = recips[0] + recips[1] + recips[2]
        p2v = p2_ref[...].reshape(S, C2)
        hi = p2v.astype(jnp.bfloat16).astype(jnp.float32)
        r1 = p2v - hi
        mid = r1.astype(jnp.bfloat16).astype(jnp.float32)
        lo = r1 - mid
        rows = []
        for s_ in sels:
            sf = s_.astype(jnp.float32)
            rows.append(
                (jnp.dot(sf, hi, preferred_element_type=jnp.float32)
                 + jnp.dot(sf, mid, preferred_element_type=jnp.float32))
                + jnp.dot(sf, lo, preferred_element_type=jnp.float32))
        interp = ((rows[0] * (recips[0] / norm)
                   + rows[1] * (recips[1] / norm))
                  + rows[2] * (recips[2] / norm))
        if C1:
            h = jnp.concatenate([p1_ref[...].reshape(n_tile, C1), interp],
                                axis=1)
        else:
            h = interp
        for j in range(nw):
            w, b, gm, bt = (w_refs[4 * j][...], w_refs[4 * j + 1][...],
                            w_refs[4 * j + 2][...], w_refs[4 * j + 3][...])
            t = jnp.dot(h, w, preferred_element_type=jnp.float32) + b
            h = jnp.maximum((t * _BN) * gm + bt, 0.0)
        o_ref[...] = h.reshape(1, n_tile, cout)

    in_specs = [pl.BlockSpec((1, n_tile, 3), lambda b, s: (b, s, 0)),
                pl.BlockSpec((1, 1, S), lambda b, s: (b, 0, 0)),
                pl.BlockSpec((1, 1, S), lambda b, s: (b, 0, 0)),
                pl.BlockSpec((1, 1, S), lambda b, s: (b, 0, 0)),
                pl.BlockSpec((1, S, C2), lambda b, s: (b, 0, 0))]
    args = [x1, x2c[0], x2c[1], x2c[2], p2]
    if C1:
        in_specs.append(pl.BlockSpec((1, n_tile, C1), lambda b, s: (b, s, 0)))
        args.append(pts1)
    for wb in wbs:
        for t_ in wb:
            in_specs.append(
                pl.BlockSpec(t_.shape, lambda *_, n=t_.ndim: (0,) * n))
            args.append(t_)
    return pl.pallas_call(
        body, grid=grid, in_specs=in_specs,
        out_specs=pl.BlockSpec((1, n_tile, cout), lambda b, s: (b, s, 0)),
        out_shape=jax.ShapeDtypeStruct((B, N, cout), jnp.float32))(*args)


def _head_call(h, p1, w2, b2, n_tile):
    B, N, C = h.shape
    ncls = w2.shape[1]
    grid = (B, N // n_tile)

    def body(h_ref, w1_ref, b1_ref, g1_ref, be1_ref, w2_ref, b2_ref, o_ref):
        hv = h_ref[...].reshape(n_tile, C)
        t = (jnp.dot(hv, w1_ref[...], preferred_element_type=jnp.float32)
             + b1_ref[...])
        hv = jnp.maximum((t * _BN) * g1_ref[...] + be1_ref[...], 0.0)
        z = (jnp.dot(hv, w2_ref[...], preferred_element_type=jnp.float32)
             + b2_ref[...])
        m = jnp.max(z, axis=1, keepdims=True)
        sh = z - m
        o = sh - jnp.log(jnp.sum(jnp.exp(sh), axis=1, keepdims=True))
        o_ref[...] = o.reshape(1, n_tile, ncls)

    in_specs = [pl.BlockSpec((1, n_tile, C), lambda b, s: (b, s, 0))]
    args = [h]
    for t_ in (*p1, w2, b2):
        in_specs.append(pl.BlockSpec(t_.shape, lambda *_: (0, 0)))
        args.append(t_)
    return pl.pallas_call(
        body, grid=grid, in_specs=in_specs,
        out_specs=pl.BlockSpec((1, n_tile, ncls), lambda b, s: (b, s, 0)),
        out_shape=jax.ShapeDtypeStruct((B, N, ncls), jnp.float32),
    )(*args)


def kernel(xyz, params):
    B, _, N = xyz.shape
    xt = jnp.transpose(xyz, (0, 2, 1))
    c0x = xyz[:, 0, :]
    c0y = xyz[:, 1, :]
    c0z = xyz[:, 2, :]
    coords0 = xt[:, :, 0:3]

    sa_w = {k: [_prep(p) for p in params[k]]
            for k in ('sa1', 'sa2', 'sa3', 'sa4')}
    fp_w = {k: [_prep(p) for p in params[k]]
            for k in ('fp4', 'fp3', 'fp2', 'fp1')}
    hp1 = _prep(params['head'][0])
    h2w = params['head'][1]['W'].T
    h2b = params['head'][1]['b'][None, :]

    def sa_level(cxa, cya, cza, coords, feats, npoint, r, st_ball, st_mlp,
                 wbs):
        nx, ny, nz = _fps_call(cxa, cya, cza, npoint)
        newc = jnp.stack([nx, ny, nz], axis=-1)
        n_cur = cxa.shape[1]
        idx = _ball_call(cxa[:, None, :], cya[:, None, :], cza[:, None, :],
                         newc, r * r, st_ball)
        fc = feats.shape[2]
        cin = 3 + fc
        dp = -(-cin // 128) * 128
        table = jnp.concatenate(
            [coords, feats, jnp.zeros((B, n_cur, dp - cin), jnp.float32)],
            axis=2)
        gidx = (idx + (jnp.arange(B, dtype=jnp.int32)[:, None, None] * n_cur)
                ).reshape(-1)
        rows = table.reshape(B * n_cur, dp)[gidx]
        g = rows.reshape(B, npoint, _NS, dp)
        h = _sa_mlp_call(g, newc, wbs, cin, st_mlp)
        return (nx, ny, nz), newc, h

    (c1x, c1y, c1z), newc1, h1 = sa_level(
        c0x, c0y, c0z, coords0, xt, 1024, 0.1, 128, 128, sa_w['sa1'])
    (c2x, c2y, c2z), newc2, h2 = sa_level(
        c1x, c1y, c1z, newc1, h1, 256, 0.2, 256, 256, sa_w['sa2'])
    (c3x, c3y, c3z), newc3, h3 = sa_level(
        c2x, c2y, c2z, newc2, h2, 64, 0.4, 64, 64, sa_w['sa3'])
    (c4x, c4y, c4z), newc4, h4 = sa_level(
        c3x, c3y, c3z, newc3, h3, 16, 0.8, 16, 16, sa_w['sa4'])

    f4 = _fp_call(newc3,
                  (c4x[:, None, :], c4y[:, None, :], c4z[:, None, :]),
                  h4, h3, fp_w['fp4'], 64)
    f3 = _fp_call(newc2,
                  (c3x[:, None, :], c3y[:, None, :], c3z[:, None, :]),
                  f4, h2, fp_w['fp3'], 256)
    f2 = _fp_call(newc1,
                  (c2x[:, None, :], c2y[:, None, :], c2z[:, None, :]),
                  f3, h1, fp_w['fp2'], 1024)
    f1 = _fp_call(coords0,
                  (c1x[:, None, :], c1y[:, None, :], c1z[:, None, :]),
                  f2, None, fp_w['fp1'], 1024)

    xout = _head_call(f1, hp1, h2w, h2b, 4096)
    l4 = jnp.transpose(h4, (0, 2, 1))
    return xout, l4
